# baseline (speedup 1.0000x reference)
import functools

import jax
import jax.numpy as jnp
from jax.experimental import pallas as pl
from jax.experimental.pallas import tpu as pltpu
from jax.experimental.pallas import tpu_sc as plsc

D_M = 1024
D_F = 4096
N_E = 8
N_TOK = 4096
CAP = max(1, int(1.25 * N_TOK / N_E))
SLOTS = N_E * CAP
PAD_ROWS = SLOTS + CAP
CHUNK = 512
DFK = 512
KC = D_F // DFK
SCW = 16


def _route_select_body(x_ref, wrt_ref, ps1_ref, ps2_ref, p1_ref, p2_ref,
                       w1_ref, w2_ref):
    x = x_ref[...]
    wrt = wrt_ref[...]
    logits = jnp.dot(x, wrt, preferred_element_type=jnp.float32)
    m = jnp.max(logits, axis=1, keepdims=True)
    ex = jnp.exp(logits - m)
    probs = ex / jnp.sum(ex, axis=1, keepdims=True)

    iota_e = jax.lax.broadcasted_iota(jnp.int32, (N_TOK, N_E), 1)
    v1 = jnp.max(probs, axis=1, keepdims=True)
    e1 = jnp.min(jnp.where(probs == v1, iota_e, N_E), axis=1, keepdims=True)
    oh1 = iota_e == e1
    probs2 = jnp.where(oh1, -1.0, probs)
    v2 = jnp.max(probs2, axis=1, keepdims=True)
    e2 = jnp.min(jnp.where(probs2 == v2, iota_e, N_E), axis=1, keepdims=True)
    oh2 = iota_e == e2
    denom = v1 + v2 + 1e-6
    w1 = v1 / denom
    w2 = v2 / denom
    wmat = jnp.where(oh1, w1, 0.0) + jnp.where(oh2, w2, 0.0)

    wbits = jax.lax.bitcast_convert_type(wmat, jnp.int32)

    def bs_body(_, carry):
        lo, hi = carry
        mid = (lo + hi) // 2
        cnt = jnp.sum((wbits >= mid).astype(jnp.int32), axis=0, keepdims=True)
        ge = cnt >= CAP
        return jnp.where(ge, mid, lo), jnp.where(ge, hi, mid)

    lo0 = jnp.zeros((1, N_E), jnp.int32)
    hi0 = jnp.full((1, N_E), 0x3F800001, jnp.int32)
    tau, _ = jax.lax.fori_loop(0, 31, bs_body, (lo0, hi0))

    gt = wbits > tau
    eq = (wbits == tau) & (tau > 0)
    n_gt = jnp.sum(gt.astype(jnp.float32), axis=0, keepdims=True)

    r_i = jax.lax.broadcasted_iota(jnp.int32, (CHUNK, CHUNK), 0)
    c_i = jax.lax.broadcasted_iota(jnp.int32, (CHUNK, CHUNK), 1)
    tri = (c_i < r_i).astype(jnp.float32)
    col_base = jax.lax.broadcasted_iota(jnp.int32, (CHUNK, N_E), 1) * CAP

    carry_eq = jnp.zeros((1, N_E), jnp.float32)
    carry_keep = jnp.zeros((1, N_E), jnp.float32)
    for c in range(N_TOK // CHUNK):
        sl = slice(c * CHUNK, (c + 1) * CHUNK)
        eq_c = eq[sl]
        gt_c = gt[sl]
        eq_f = eq_c.astype(jnp.float32)
        pe = jnp.dot(tri, eq_f, preferred_element_type=jnp.float32) + carry_eq
        keep_eq = eq_c & ((n_gt + pe) < CAP)
        keep_c = gt_c | keep_eq
        keep_f = keep_c.astype(jnp.float32)
        pk = jnp.dot(tri, keep_f, preferred_element_type=jnp.float32) + carry_keep
        slot_c = pk.astype(jnp.int32)
        pos_c = jnp.where(keep_c, col_base + slot_c, SLOTS)

        oh1_c = oh1[sl]
        oh2_c = oh2[sl]
        zero = jnp.zeros_like(pos_c)
        ps1_c = jnp.sum(jnp.where(oh1_c, pos_c, zero), axis=1, keepdims=True)
        ps2_c = jnp.sum(jnp.where(oh2_c, pos_c, zero), axis=1, keepdims=True)
        k1_c = ps1_c < SLOTS
        k2_c = ps2_c < SLOTS
        ps1_ref[sl, :] = ps1_c
        ps2_ref[sl, :] = ps2_c
        p1_ref[sl, :] = jnp.where(k1_c, ps1_c, 0)
        p2_ref[sl, :] = jnp.where(k2_c, ps2_c, 0)
        w1_ref[sl, :] = jnp.where(k1_c, w1[sl], 0.0)
        w2_ref[sl, :] = jnp.where(k2_c, w2[sl], 0.0)

        carry_eq = carry_eq + jnp.sum(eq_f, axis=0, keepdims=True)
        carry_keep = carry_keep + jnp.sum(keep_f, axis=0, keepdims=True)


def _ffn_body(xin_ref, wg_ref, wu_ref, wd_ref, y_ref):
    k = pl.program_id(1)
    xin = xin_ref[...]
    xin = jnp.where(xin != xin, 0.0, xin)
    xb = jnp.clip(xin, -1e4, 1e4).astype(jnp.bfloat16)
    wg = wg_ref[0].astype(jnp.bfloat16)
    wu = wu_ref[0].astype(jnp.bfloat16)
    wd = wd_ref[0].astype(jnp.bfloat16)
    gate = jnp.dot(xb, wg, preferred_element_type=jnp.float32)
    up = jnp.dot(xb, wu, preferred_element_type=jnp.float32)
    h = (gate * jax.nn.sigmoid(gate) * up).astype(jnp.bfloat16)
    yc = jnp.dot(h, wd, preferred_element_type=jnp.float32)

    @pl.when(k == 0)
    def _():
        y_ref[...] = yc

    @pl.when(k != 0)
    def _():
        y_ref[...] += yc


def _combine_body(g1_ref, g2_ref, w1_ref, w2_ref, o_ref):
    o_ref[...] = w1_ref[...] * g1_ref[...] + w2_ref[...] * g2_ref[...]


def _route_select(x_flat, wrt):
    i32 = jnp.int32
    f32 = jnp.float32
    return pl.pallas_call(
        _route_select_body,
        out_shape=[
            jax.ShapeDtypeStruct((N_TOK, 1), i32),
            jax.ShapeDtypeStruct((N_TOK, 1), i32),
            jax.ShapeDtypeStruct((N_TOK, 1), i32),
            jax.ShapeDtypeStruct((N_TOK, 1), i32),
            jax.ShapeDtypeStruct((N_TOK, 1), f32),
            jax.ShapeDtypeStruct((N_TOK, 1), f32),
        ],
    )(x_flat, wrt)


def _sc_mesh():
    return plsc.VectorSubcoreMesh(core_axis_name="c", subcore_axis_name="s")


_N_WORKERS = 32
_PER_W = N_TOK // _N_WORKERS
_J = _PER_W // SCW


def _dispatch(x_flat, i1_flat, i2_flat):
    @pl.kernel(
        out_type=jax.ShapeDtypeStruct((PAD_ROWS, D_M), jnp.float32),
        mesh=_sc_mesh(),
        scratch_types=[
            pltpu.VMEM((SCW,), jnp.int32),
            pltpu.VMEM((SCW,), jnp.int32),
            pltpu.VMEM((SCW, D_M), jnp.float32),
        ],
    )
    def run(x_hbm, i1_hbm, i2_hbm, o_hbm, i1_v, i2_v, rows_v):
        wid = jax.lax.axis_index("s") * 2 + jax.lax.axis_index("c")

        @pl.loop(0, _J)
        def _(j):
            base = wid * _PER_W + j * SCW
            pltpu.sync_copy(x_hbm.at[pl.ds(base, SCW)], rows_v)
            pltpu.sync_copy(i1_hbm.at[pl.ds(base, SCW)], i1_v)
            pltpu.sync_copy(rows_v, o_hbm.at[i1_v])
            pltpu.sync_copy(i2_hbm.at[pl.ds(base, SCW)], i2_v)
            pltpu.sync_copy(rows_v, o_hbm.at[i2_v])

    return run(x_flat, i1_flat, i2_flat)


def _combine_gather(y, i1_flat, i2_flat):
    @pl.kernel(
        out_type=[
            jax.ShapeDtypeStruct((N_TOK, D_M), jnp.float32),
            jax.ShapeDtypeStruct((N_TOK, D_M), jnp.float32),
        ],
        mesh=_sc_mesh(),
        scratch_types=[
            pltpu.VMEM((SCW,), jnp.int32),
            pltpu.VMEM((SCW,), jnp.int32),
            pltpu.VMEM((SCW, D_M), jnp.float32),
            pltpu.VMEM((SCW, D_M), jnp.float32),
        ],
    )
    def run(y_hbm, i1_hbm, i2_hbm, g1_hbm, g2_hbm, i1_v, i2_v, r1_v, r2_v):
        wid = jax.lax.axis_index("s") * 2 + jax.lax.axis_index("c")

        @pl.loop(0, _J)
        def _(j):
            base = wid * _PER_W + j * SCW
            pltpu.sync_copy(i1_hbm.at[pl.ds(base, SCW)], i1_v)
            pltpu.sync_copy(y_hbm.at[i1_v], r1_v)
            pltpu.sync_copy(r1_v, g1_hbm.at[pl.ds(base, SCW)])
            pltpu.sync_copy(i2_hbm.at[pl.ds(base, SCW)], i2_v)
            pltpu.sync_copy(y_hbm.at[i2_v], r2_v)
            pltpu.sync_copy(r2_v, g2_hbm.at[pl.ds(base, SCW)])

    return run(y, i1_flat, i2_flat)


def _ffn(exp_in, Wg, Wu, Wd):
    return pl.pallas_call(
        _ffn_body,
        grid=(N_E, KC),
        in_specs=[
            pl.BlockSpec((CAP, D_M), lambda e, k: (e, 0)),
            pl.BlockSpec((1, D_M, DFK), lambda e, k: (e, 0, k)),
            pl.BlockSpec((1, D_M, DFK), lambda e, k: (e, 0, k)),
            pl.BlockSpec((1, DFK, D_M), lambda e, k: (e, k, 0)),
        ],
        out_specs=pl.BlockSpec((CAP, D_M), lambda e, k: (e, 0)),
        out_shape=jax.ShapeDtypeStruct((SLOTS, D_M), jnp.float32),
    )(exp_in, Wg, Wu, Wd)


def _combine(g1, g2, w1, w2):
    blk = 512
    return pl.pallas_call(
        _combine_body,
        grid=(N_TOK // blk,),
        in_specs=[
            pl.BlockSpec((blk, D_M), lambda i: (i, 0)),
            pl.BlockSpec((blk, D_M), lambda i: (i, 0)),
            pl.BlockSpec((blk, 1), lambda i: (i, 0)),
            pl.BlockSpec((blk, 1), lambda i: (i, 0)),
        ],
        out_specs=pl.BlockSpec((blk, D_M), lambda i: (i, 0)),
        out_shape=jax.ShapeDtypeStruct((N_TOK, D_M), jnp.float32),
    )(g1, g2, w1, w2)


@jax.jit
def kernel(x, Wr, Wg, Wu, Wd):
    B, L, D = x.shape
    assert (B * L, D) == (N_TOK, D_M)
    x_flat = x.reshape(N_TOK, D_M)
    wrt = Wr.T

    ps1, ps2, p1, p2, w1, w2 = _route_select(x_flat, wrt)

    exp_in = _dispatch(x_flat, ps1.reshape(N_TOK), ps2.reshape(N_TOK))
    y = _ffn(exp_in, Wg, Wu, Wd)
    g1, g2 = _combine_gather(y, p1.reshape(N_TOK), p2.reshape(N_TOK))
    out = _combine(g1, g2, w1, w2)
    return out.reshape(B, L, D)


# speedup vs baseline: 1.0481x; 1.0481x over previous
import functools

import jax
import jax.numpy as jnp
from jax.experimental import pallas as pl
from jax.experimental.pallas import tpu as pltpu
from jax.experimental.pallas import tpu_sc as plsc

D_M = 1024
D_F = 4096
N_E = 8
N_TOK = 4096
CAP = max(1, int(1.25 * N_TOK / N_E))
SLOTS = N_E * CAP
PAD_ROWS = SLOTS + CAP
CHUNK = 512
DFK = 512
KC = D_F // DFK
SCW = 64


def _route_select_body(x_ref, wrt_ref, ps1_ref, ps2_ref, p1_ref, p2_ref,
                       w1_ref, w2_ref):
    x = x_ref[...]
    wrt = wrt_ref[...]
    logits = jnp.dot(x, wrt, preferred_element_type=jnp.float32)
    m = jnp.max(logits, axis=1, keepdims=True)
    ex = jnp.exp(logits - m)
    probs = ex / jnp.sum(ex, axis=1, keepdims=True)

    iota_e = jax.lax.broadcasted_iota(jnp.int32, (N_TOK, N_E), 1)
    v1 = jnp.max(probs, axis=1, keepdims=True)
    e1 = jnp.min(jnp.where(probs == v1, iota_e, N_E), axis=1, keepdims=True)
    oh1 = iota_e == e1
    probs2 = jnp.where(oh1, -1.0, probs)
    v2 = jnp.max(probs2, axis=1, keepdims=True)
    e2 = jnp.min(jnp.where(probs2 == v2, iota_e, N_E), axis=1, keepdims=True)
    oh2 = iota_e == e2
    denom = v1 + v2 + 1e-6
    w1 = v1 / denom
    w2 = v2 / denom
    wmat = jnp.where(oh1, w1, 0.0) + jnp.where(oh2, w2, 0.0)

    wbits = jax.lax.bitcast_convert_type(wmat, jnp.int32)

    def bs_body(_, carry):
        lo, hi = carry
        mid = (lo + hi) // 2
        cnt = jnp.sum((wbits >= mid).astype(jnp.int32), axis=0, keepdims=True)
        ge = cnt >= CAP
        return jnp.where(ge, mid, lo), jnp.where(ge, hi, mid)

    lo0 = jnp.zeros((1, N_E), jnp.int32)
    hi0 = jnp.full((1, N_E), 0x3F800001, jnp.int32)
    tau, _ = jax.lax.fori_loop(0, 31, bs_body, (lo0, hi0))

    gt = wbits > tau
    eq = (wbits == tau) & (tau > 0)
    n_gt = jnp.sum(gt.astype(jnp.float32), axis=0, keepdims=True)

    r_i = jax.lax.broadcasted_iota(jnp.int32, (CHUNK, CHUNK), 0)
    c_i = jax.lax.broadcasted_iota(jnp.int32, (CHUNK, CHUNK), 1)
    tri = (c_i < r_i).astype(jnp.float32)
    col_base = jax.lax.broadcasted_iota(jnp.int32, (CHUNK, N_E), 1) * CAP

    carry_eq = jnp.zeros((1, N_E), jnp.float32)
    carry_keep = jnp.zeros((1, N_E), jnp.float32)
    for c in range(N_TOK // CHUNK):
        sl = slice(c * CHUNK, (c + 1) * CHUNK)
        eq_c = eq[sl]
        gt_c = gt[sl]
        eq_f = eq_c.astype(jnp.float32)
        pe = jnp.dot(tri, eq_f, preferred_element_type=jnp.float32) + carry_eq
        keep_eq = eq_c & ((n_gt + pe) < CAP)
        keep_c = gt_c | keep_eq
        keep_f = keep_c.astype(jnp.float32)
        pk = jnp.dot(tri, keep_f, preferred_element_type=jnp.float32) + carry_keep
        slot_c = pk.astype(jnp.int32)
        pos_c = jnp.where(keep_c, col_base + slot_c, SLOTS)

        oh1_c = oh1[sl]
        oh2_c = oh2[sl]
        zero = jnp.zeros_like(pos_c)
        ps1_c = jnp.sum(jnp.where(oh1_c, pos_c, zero), axis=1, keepdims=True)
        ps2_c = jnp.sum(jnp.where(oh2_c, pos_c, zero), axis=1, keepdims=True)
        k1_c = ps1_c < SLOTS
        k2_c = ps2_c < SLOTS
        ps1_ref[sl, :] = ps1_c
        ps2_ref[sl, :] = ps2_c
        p1_ref[sl, :] = jnp.where(k1_c, ps1_c, 0)
        p2_ref[sl, :] = jnp.where(k2_c, ps2_c, 0)
        w1_ref[sl, :] = jnp.where(k1_c, w1[sl], 0.0)
        w2_ref[sl, :] = jnp.where(k2_c, w2[sl], 0.0)

        carry_eq = carry_eq + jnp.sum(eq_f, axis=0, keepdims=True)
        carry_keep = carry_keep + jnp.sum(keep_f, axis=0, keepdims=True)


def _ffn_body(xin_ref, wg_ref, wu_ref, wd_ref, y_ref):
    k = pl.program_id(1)
    xin = xin_ref[...]
    xin = jnp.where(xin != xin, 0.0, xin)
    xb = jnp.clip(xin, -1e4, 1e4).astype(jnp.bfloat16)
    wg = wg_ref[0].astype(jnp.bfloat16)
    wu = wu_ref[0].astype(jnp.bfloat16)
    wd = wd_ref[0].astype(jnp.bfloat16)
    gate = jnp.dot(xb, wg, preferred_element_type=jnp.float32)
    up = jnp.dot(xb, wu, preferred_element_type=jnp.float32)
    h = (gate * jax.nn.sigmoid(gate) * up).astype(jnp.bfloat16)
    yc = jnp.dot(h, wd, preferred_element_type=jnp.float32)

    @pl.when(k == 0)
    def _():
        y_ref[...] = yc

    @pl.when(k != 0)
    def _():
        y_ref[...] += yc


def _combine_body(g1_ref, g2_ref, w1_ref, w2_ref, o_ref):
    o_ref[...] = w1_ref[...] * g1_ref[...] + w2_ref[...] * g2_ref[...]


def _route_select(x_flat, wrt):
    i32 = jnp.int32
    f32 = jnp.float32
    return pl.pallas_call(
        _route_select_body,
        out_shape=[
            jax.ShapeDtypeStruct((N_TOK, 1), i32),
            jax.ShapeDtypeStruct((N_TOK, 1), i32),
            jax.ShapeDtypeStruct((N_TOK, 1), i32),
            jax.ShapeDtypeStruct((N_TOK, 1), i32),
            jax.ShapeDtypeStruct((N_TOK, 1), f32),
            jax.ShapeDtypeStruct((N_TOK, 1), f32),
        ],
    )(x_flat, wrt)


def _sc_mesh():
    return plsc.VectorSubcoreMesh(core_axis_name="c", subcore_axis_name="s")


_N_WORKERS = 32
_PER_W = N_TOK // _N_WORKERS
_J = _PER_W // SCW


def _dispatch(x_flat, i1_flat, i2_flat):
    @pl.kernel(
        out_type=jax.ShapeDtypeStruct((PAD_ROWS, D_M), jnp.float32),
        mesh=_sc_mesh(),
        scratch_types=[
            pltpu.VMEM((SCW,), jnp.int32),
            pltpu.VMEM((SCW,), jnp.int32),
            pltpu.VMEM((SCW, D_M), jnp.float32),
        ],
    )
    def run(x_hbm, i1_hbm, i2_hbm, o_hbm, i1_v, i2_v, rows_v):
        wid = jax.lax.axis_index("s") * 2 + jax.lax.axis_index("c")

        @pl.loop(0, _J)
        def _(j):
            base = wid * _PER_W + j * SCW
            pltpu.sync_copy(x_hbm.at[pl.ds(base, SCW)], rows_v)
            pltpu.sync_copy(i1_hbm.at[pl.ds(base, SCW)], i1_v)
            pltpu.sync_copy(rows_v, o_hbm.at[i1_v])
            pltpu.sync_copy(i2_hbm.at[pl.ds(base, SCW)], i2_v)
            pltpu.sync_copy(rows_v, o_hbm.at[i2_v])

    return run(x_flat, i1_flat, i2_flat)


def _combine_gather(y, i1_flat, i2_flat):
    @pl.kernel(
        out_type=[
            jax.ShapeDtypeStruct((N_TOK, D_M), jnp.float32),
            jax.ShapeDtypeStruct((N_TOK, D_M), jnp.float32),
        ],
        mesh=_sc_mesh(),
        scratch_types=[
            pltpu.VMEM((SCW,), jnp.int32),
            pltpu.VMEM((SCW,), jnp.int32),
            pltpu.VMEM((SCW, D_M), jnp.float32),
        ],
    )
    def run(y_hbm, i1_hbm, i2_hbm, g1_hbm, g2_hbm, i1_v, i2_v, r_v):
        wid = jax.lax.axis_index("s") * 2 + jax.lax.axis_index("c")

        @pl.loop(0, _J)
        def _(j):
            base = wid * _PER_W + j * SCW
            pltpu.sync_copy(i1_hbm.at[pl.ds(base, SCW)], i1_v)
            pltpu.sync_copy(y_hbm.at[i1_v], r_v)
            pltpu.sync_copy(r_v, g1_hbm.at[pl.ds(base, SCW)])
            pltpu.sync_copy(i2_hbm.at[pl.ds(base, SCW)], i2_v)
            pltpu.sync_copy(y_hbm.at[i2_v], r_v)
            pltpu.sync_copy(r_v, g2_hbm.at[pl.ds(base, SCW)])

    return run(y, i1_flat, i2_flat)


def _ffn(exp_in, Wg, Wu, Wd):
    return pl.pallas_call(
        _ffn_body,
        grid=(N_E, KC),
        in_specs=[
            pl.BlockSpec((CAP, D_M), lambda e, k: (e, 0)),
            pl.BlockSpec((1, D_M, DFK), lambda e, k: (e, 0, k)),
            pl.BlockSpec((1, D_M, DFK), lambda e, k: (e, 0, k)),
            pl.BlockSpec((1, DFK, D_M), lambda e, k: (e, k, 0)),
        ],
        out_specs=pl.BlockSpec((CAP, D_M), lambda e, k: (e, 0)),
        out_shape=jax.ShapeDtypeStruct((SLOTS, D_M), jnp.float32),
        compiler_params=pltpu.CompilerParams(
            dimension_semantics=("parallel", "arbitrary")),
    )(exp_in, Wg, Wu, Wd)


def _combine(g1, g2, w1, w2):
    blk = 512
    return pl.pallas_call(
        _combine_body,
        grid=(N_TOK // blk,),
        in_specs=[
            pl.BlockSpec((blk, D_M), lambda i: (i, 0)),
            pl.BlockSpec((blk, D_M), lambda i: (i, 0)),
            pl.BlockSpec((blk, 1), lambda i: (i, 0)),
            pl.BlockSpec((blk, 1), lambda i: (i, 0)),
        ],
        out_specs=pl.BlockSpec((blk, D_M), lambda i: (i, 0)),
        out_shape=jax.ShapeDtypeStruct((N_TOK, D_M), jnp.float32),
    )(g1, g2, w1, w2)


@jax.jit
def kernel(x, Wr, Wg, Wu, Wd):
    B, L, D = x.shape
    assert (B * L, D) == (N_TOK, D_M)
    x_flat = x.reshape(N_TOK, D_M)
    wrt = Wr.T

    ps1, ps2, p1, p2, w1, w2 = _route_select(x_flat, wrt)

    exp_in = _dispatch(x_flat, ps1.reshape(N_TOK), ps2.reshape(N_TOK))
    y = _ffn(exp_in, Wg, Wu, Wd)
    g1, g2 = _combine_gather(y, p1.reshape(N_TOK), p2.reshape(N_TOK))
    out = _combine(g1, g2, w1, w2)
    return out.reshape(B, L, D)
